# baseline (speedup 1.0000x reference)
import functools

import jax
import jax.numpy as jnp
from jax import lax
from jax.experimental import pallas as pl
from jax.experimental.pallas import tpu as pltpu
from jax.experimental.pallas import tpu_sc as plsc

N = 10000
D = 128
E = 320000
NC = 2
NS = 16
NW = NC * NS

NPAD = 10240
ROWS_PER_TILE = NPAD // NS
K = 48
NBUF = 4
NCHUNK = 212
EPT = K * NCHUNK
EPAD = NW * EPT
ZK = 80

def _hist_body(dst_hbm, z1d_hbm, out_hbm, idx_v, hist_v, hsum, tb, res_v):
    cid = lax.axis_index("c")
    sid = lax.axis_index("s")
    wid = cid * NS + sid

    pltpu.sync_copy(dst_hbm.at[wid], idx_v)
    pltpu.sync_copy(z1d_hbm, hist_v)

    ones = jnp.full((16,), 1.0, jnp.float32)

    def chunk(j, _):
        for k in range(K // 16):
            idx = idx_v[j, pl.ds(k * 16, 16)]
            plsc.addupdate_scatter(hist_v, [idx], ones)
        return 0

    lax.fori_loop(0, NCHUNK, chunk, 0)

    pltpu.sync_copy(hist_v, hsum.at[sid])
    plsc.subcore_barrier()

    base = sid * ROWS_PER_TILE
    pltpu.sync_copy(hsum.at[:, pl.ds(base, ROWS_PER_TILE)], tb)

    def red(c, _):
        acc = tb[0, pl.ds(c * 16, 16)]
        for t in range(1, NS):
            acc = acc + tb[t, pl.ds(c * 16, 16)]
        res_v[pl.ds(c * 16, 16)] = acc
        return 0

    lax.fori_loop(0, ROWS_PER_TILE // 16, red, 0)
    pltpu.sync_copy(res_v, out_hbm.at[cid, pl.ds(base, ROWS_PER_TILE)])


@functools.cache
def _hist_kernel():
    mesh = plsc.VectorSubcoreMesh(core_axis_name="c", subcore_axis_name="s")
    return pl.kernel(
        _hist_body,
        out_type=jax.ShapeDtypeStruct((NC, NPAD), jnp.float32),
        mesh=mesh,
        compiler_params=pltpu.CompilerParams(needs_layout_passes=False, use_tc_tiling_on_sc=False),
        scratch_types=[
            pltpu.VMEM((NCHUNK, K), jnp.int32),
            pltpu.VMEM((NPAD,), jnp.float32),
            pltpu.VMEM_SHARED((NS, NPAD), jnp.float32),
            pltpu.VMEM((NS, ROWS_PER_TILE), jnp.float32),
            pltpu.VMEM((ROWS_PER_TILE,), jnp.float32),
        ],
    )


def _edge_body(y_hbm, src_hbm, dst_hbm, z2d_hbm, out_hbm,
               src_v, dst_v, rows, acc, gsems, ssems):
    cid = lax.axis_index("c")
    sid = lax.axis_index("s")
    wid = cid * NS + sid

    pltpu.sync_copy(src_hbm.at[wid], src_v)
    pltpu.sync_copy(dst_hbm.at[wid], dst_v)

    base = sid * ROWS_PER_TILE
    for s in range(ROWS_PER_TILE // ZK):
        pltpu.sync_copy(z2d_hbm, acc.at[pl.ds(base + s * ZK, ZK)])
    plsc.subcore_barrier()

    def gather(j, b):
        return pltpu.async_copy(y_hbm.at[src_v.at[j]], rows.at[b], gsems.at[b])

    def scatter(j, b):
        return pltpu.async_copy(rows.at[b], acc.at[dst_v.at[j]], ssems.at[b],
                                add=True)

    gather(0, 0)
    gather(1, 1)

    def step(i, _):
        for b in range(NBUF):
            j = NBUF * i + b
            pltpu.make_async_copy(y_hbm.at[src_v.at[j]], rows.at[b],
                                  gsems.at[b]).wait()
            scatter(j, b)
            b2 = (b + 2) % NBUF

            @pl.when(j >= 2)
            def _():
                pltpu.make_async_copy(rows.at[b2], acc.at[dst_v.at[j - 2]],
                                      ssems.at[b2]).wait()

            @pl.when(j + 2 < NCHUNK)
            def _():
                gather(j + 2, b2)
        return 0

    lax.fori_loop(0, NCHUNK // NBUF, step, 0)
    pltpu.make_async_copy(rows.at[2], acc.at[dst_v.at[NCHUNK - 2]],
                          ssems.at[2]).wait()
    pltpu.make_async_copy(rows.at[3], acc.at[dst_v.at[NCHUNK - 1]],
                          ssems.at[3]).wait()
    plsc.subcore_barrier()
    pltpu.sync_copy(acc.at[pl.ds(base, ROWS_PER_TILE)],
                    out_hbm.at[cid, pl.ds(base, ROWS_PER_TILE)])


@functools.cache
def _edge_kernel():
    mesh = plsc.VectorSubcoreMesh(core_axis_name="c", subcore_axis_name="s")
    return pl.kernel(
        _edge_body,
        out_type=jax.ShapeDtypeStruct((NC, NPAD, D), jnp.float32),
        mesh=mesh,
        compiler_params=pltpu.CompilerParams(needs_layout_passes=False, use_tc_tiling_on_sc=False),
        scratch_types=[
            pltpu.VMEM((NCHUNK, K), jnp.int32),
            pltpu.VMEM((NCHUNK, K), jnp.int32),
            pltpu.VMEM((NBUF, K, D), jnp.float32),
            pltpu.VMEM_SHARED((NPAD, D), jnp.float32),
            pltpu.SemaphoreType.DMA((NBUF,)),
            pltpu.SemaphoreType.DMA((NBUF,)),
        ],
    )


def _prep_body(state_ref, w_ref, hist_ref, y_ref, dinv_ref):
    deg = hist_ref[0] + hist_ref[1] + 1.0
    dinv = lax.rsqrt(deg)[:N]
    dinv_ref[...] = dinv
    xw = jnp.dot(state_ref[...], w_ref[...], preferred_element_type=jnp.float32)
    y_ref[...] = xw * dinv


def _tc_prep(state, w_conv, hist3):
    return pl.pallas_call(
        _prep_body,
        out_shape=[
            jax.ShapeDtypeStruct((N, D), jnp.float32),
            jax.ShapeDtypeStruct((N, 1), jnp.float32),
        ],
    )(state, w_conv, hist3)


def _final_body(agg_ref, y_ref, dinv_ref, state_ref, bc_ref,
                w1_ref, b1_ref, w2_ref, b2_ref, w3_ref, b3_ref, out_ref):
    agg = agg_ref[0, :N, :] + agg_ref[1, :N, :]
    conv = dinv_ref[...] * (agg + y_ref[...]) + bc_ref[...]
    h = jnp.maximum(conv, 0.0) + state_ref[...]
    z = jnp.dot(h, w1_ref[...], preferred_element_type=jnp.float32) + b1_ref[...]
    z = jnp.where(z >= 0.0, z, 0.01 * z)
    z = jnp.dot(z, w2_ref[...], preferred_element_type=jnp.float32) + b2_ref[...]
    z = jnp.where(z >= 0.0, z, 0.01 * z)
    t = jnp.sum(z * w3_ref[...], axis=1, keepdims=True) + b3_ref[...]
    c = jnp.maximum(t, 0.0) + jnp.log1p(jnp.exp(-jnp.abs(t)))
    out_ref[...] = c / (jnp.sum(c) + 1e-20)


def _tc_final(agg2, y, dinv, state, b_conv, W1, b1, W2, b2, w3r, b3):
    return pl.pallas_call(
        _final_body,
        out_shape=jax.ShapeDtypeStruct((N, 1), jnp.float32),
    )(agg2, y, dinv, state, b_conv, W1, b1, W2, b2, w3r, b3)


def kernel(state, edge_index, W_conv, b_conv, W1, b1, W2, b2, W3, b3,
           deterministic=True):
    npad_e = EPAD - E
    src = jnp.concatenate(
        [edge_index[0], jnp.zeros((npad_e,), jnp.int32)]).reshape(NW, NCHUNK, K)
    dst = jnp.concatenate(
        [edge_index[1], jnp.full((npad_e,), NPAD - 1, jnp.int32)]
    ).reshape(NW, NCHUNK, K)

    z1d = jnp.zeros((NPAD,), jnp.float32)
    z2d = jnp.zeros((ZK, D), jnp.float32)

    hist = _hist_kernel()(dst, z1d)
    hist3 = hist.reshape(NC, NPAD, 1)
    y, dinv = _tc_prep(state, W_conv, hist3)
    agg2 = _edge_kernel()(y, src, dst, z2d)
    action = _tc_final(agg2, y, dinv, state,
                       b_conv.reshape(1, D),
                       W1, b1.reshape(1, -1), W2, b2.reshape(1, -1),
                       W3.reshape(1, -1), b3.reshape(1, 1))
    return action.reshape(N // 10, 10)


# speedup vs baseline: 1.3310x; 1.3310x over previous
import functools

import jax
import jax.numpy as jnp
from jax import lax
from jax.experimental import pallas as pl
from jax.experimental.pallas import tpu as pltpu
from jax.experimental.pallas import tpu_sc as plsc

N = 10000
D = 128
E = 320000
NC = 2
NS = 16
NW = NC * NS

NPAD = 10240
ROWS_PER_TILE = NPAD // NS
K = 112
NCHUNK = 90
EPT = K * NCHUNK
EPAD = NW * EPT
ZK = 160

def _hist_body(dst_hbm, z1d_hbm, out_hbm, idx_v, hist_v, hsum, tb, res_v):
    cid = lax.axis_index("c")
    sid = lax.axis_index("s")
    wid = cid * NS + sid

    pltpu.sync_copy(dst_hbm.at[wid], idx_v)
    pltpu.sync_copy(z1d_hbm, hist_v)

    ones = jnp.full((16,), 1.0, jnp.float32)

    def chunk(j, _):
        for k in range(K // 16):
            idx = idx_v[j, pl.ds(k * 16, 16)]
            plsc.addupdate_scatter(hist_v, [idx], ones)
        return 0

    lax.fori_loop(0, NCHUNK, chunk, 0)

    pltpu.sync_copy(hist_v, hsum.at[sid])
    plsc.subcore_barrier()

    base = sid * ROWS_PER_TILE
    pltpu.sync_copy(hsum.at[:, pl.ds(base, ROWS_PER_TILE)], tb)

    def red(c, _):
        acc = tb[0, pl.ds(c * 16, 16)]
        for t in range(1, NS):
            acc = acc + tb[t, pl.ds(c * 16, 16)]
        res_v[pl.ds(c * 16, 16)] = acc
        return 0

    lax.fori_loop(0, ROWS_PER_TILE // 16, red, 0)
    pltpu.sync_copy(res_v, out_hbm.at[cid, pl.ds(base, ROWS_PER_TILE)])


@functools.cache
def _hist_kernel():
    mesh = plsc.VectorSubcoreMesh(core_axis_name="c", subcore_axis_name="s")
    return pl.kernel(
        _hist_body,
        out_type=jax.ShapeDtypeStruct((NC, NPAD), jnp.float32),
        mesh=mesh,
        compiler_params=pltpu.CompilerParams(needs_layout_passes=False, use_tc_tiling_on_sc=False),
        scratch_types=[
            pltpu.VMEM((NCHUNK, K), jnp.int32),
            pltpu.VMEM((NPAD,), jnp.float32),
            pltpu.VMEM_SHARED((NS, NPAD), jnp.float32),
            pltpu.VMEM((NS, ROWS_PER_TILE), jnp.float32),
            pltpu.VMEM((ROWS_PER_TILE,), jnp.float32),
        ],
    )


def _edge_body(y_hbm, src_hbm, dst_hbm, z2d_hbm, out_hbm,
               src_v, dst_v, rows, acc, gsems):
    cid = lax.axis_index("c")
    sid = lax.axis_index("s")
    wid = cid * NS + sid

    pltpu.sync_copy(src_hbm.at[wid], src_v)
    pltpu.sync_copy(dst_hbm.at[wid], dst_v)

    base = sid * ROWS_PER_TILE
    for s in range(ROWS_PER_TILE // ZK):
        pltpu.sync_copy(z2d_hbm, acc.at[pl.ds(base + s * ZK, ZK)])
    plsc.subcore_barrier()

    pltpu.async_copy(y_hbm.at[src_v.at[0]], rows.at[0], gsems.at[0])

    def step(i, _):
        j = 2 * i
        pltpu.make_async_copy(y_hbm.at[src_v.at[j]], rows.at[0],
                              gsems.at[0]).wait()
        pltpu.async_copy(y_hbm.at[src_v.at[j + 1]], rows.at[1], gsems.at[1])
        pltpu.sync_copy(rows.at[0], acc.at[dst_v.at[j]], add=True)
        pltpu.make_async_copy(y_hbm.at[src_v.at[j + 1]], rows.at[1],
                              gsems.at[1]).wait()

        @pl.when(i < NCHUNK // 2 - 1)
        def _():
            pltpu.async_copy(y_hbm.at[src_v.at[j + 2]], rows.at[0], gsems.at[0])

        pltpu.sync_copy(rows.at[1], acc.at[dst_v.at[j + 1]], add=True)
        return 0

    lax.fori_loop(0, NCHUNK // 2, step, 0)
    plsc.subcore_barrier()
    pltpu.sync_copy(acc.at[pl.ds(base, ROWS_PER_TILE)],
                    out_hbm.at[cid, pl.ds(base, ROWS_PER_TILE)])


@functools.cache
def _edge_kernel():
    mesh = plsc.VectorSubcoreMesh(core_axis_name="c", subcore_axis_name="s")
    return pl.kernel(
        _edge_body,
        out_type=jax.ShapeDtypeStruct((NC, NPAD, D), jnp.float32),
        mesh=mesh,
        compiler_params=pltpu.CompilerParams(needs_layout_passes=False, use_tc_tiling_on_sc=False),
        scratch_types=[
            pltpu.VMEM((NCHUNK, K), jnp.int32),
            pltpu.VMEM((NCHUNK, K), jnp.int32),
            pltpu.VMEM((2, K, D), jnp.float32),
            pltpu.VMEM_SHARED((NPAD, D), jnp.float32),
            pltpu.SemaphoreType.DMA((2,)),
        ],
    )


def _prep_body(state_ref, w_ref, hist_ref, y_ref, dinv_ref):
    deg = hist_ref[0] + hist_ref[1] + 1.0
    dinv = lax.rsqrt(deg)[:N]
    dinv_ref[...] = dinv
    xw = jnp.dot(state_ref[...], w_ref[...], preferred_element_type=jnp.float32)
    y_ref[...] = xw * dinv


def _tc_prep(state, w_conv, hist3):
    return pl.pallas_call(
        _prep_body,
        out_shape=[
            jax.ShapeDtypeStruct((N, D), jnp.float32),
            jax.ShapeDtypeStruct((N, 1), jnp.float32),
        ],
    )(state, w_conv, hist3)


def _final_body(agg_ref, y_ref, dinv_ref, state_ref, bc_ref,
                w1_ref, b1_ref, w2_ref, b2_ref, w3_ref, b3_ref, out_ref):
    agg = agg_ref[0, :N, :] + agg_ref[1, :N, :]
    conv = dinv_ref[...] * (agg + y_ref[...]) + bc_ref[...]
    h = jnp.maximum(conv, 0.0) + state_ref[...]
    z = jnp.dot(h, w1_ref[...], preferred_element_type=jnp.float32) + b1_ref[...]
    z = jnp.where(z >= 0.0, z, 0.01 * z)
    z = jnp.dot(z, w2_ref[...], preferred_element_type=jnp.float32) + b2_ref[...]
    z = jnp.where(z >= 0.0, z, 0.01 * z)
    t = jnp.sum(z * w3_ref[...], axis=1, keepdims=True) + b3_ref[...]
    c = jnp.maximum(t, 0.0) + jnp.log1p(jnp.exp(-jnp.abs(t)))
    out_ref[...] = c / (jnp.sum(c) + 1e-20)


def _tc_final(agg2, y, dinv, state, b_conv, W1, b1, W2, b2, w3r, b3):
    return pl.pallas_call(
        _final_body,
        out_shape=jax.ShapeDtypeStruct((N, 1), jnp.float32),
    )(agg2, y, dinv, state, b_conv, W1, b1, W2, b2, w3r, b3)


def kernel(state, edge_index, W_conv, b_conv, W1, b1, W2, b2, W3, b3,
           deterministic=True):
    npad_e = EPAD - E
    src = jnp.concatenate(
        [edge_index[0], jnp.zeros((npad_e,), jnp.int32)]).reshape(NW, NCHUNK, K)
    dst = jnp.concatenate(
        [edge_index[1], jnp.full((npad_e,), NPAD - 1, jnp.int32)]
    ).reshape(NW, NCHUNK, K)

    z1d = jnp.zeros((NPAD,), jnp.float32)
    z2d = jnp.zeros((ZK, D), jnp.float32)

    hist = _hist_kernel()(dst, z1d)
    hist3 = hist.reshape(NC, NPAD, 1)
    y, dinv = _tc_prep(state, W_conv, hist3)
    agg2 = _edge_kernel()(y, src, dst, z2d)
    action = _tc_final(agg2, y, dinv, state,
                       b_conv.reshape(1, D),
                       W1, b1.reshape(1, -1), W2, b2.reshape(1, -1),
                       W3.reshape(1, -1), b3.reshape(1, 1))
    return action.reshape(N // 10, 10)
